# baseline (speedup 1.0000x reference)
import functools

import jax
import jax.numpy as jnp
from jax import lax
from jax.experimental import pallas as pl
from jax.experimental.pallas import tpu as pltpu
from jax.experimental.pallas import tpu_sc as plsc

EMB = 256
HEADS = 8
DH = EMB // HEADS
NNB = 16
N = 10000

P = 5
CN = N // P
EC = CN * NNB

NS = 16
CHUNK = 128
NCH = 16
EPAD = NS * NCH * CHUNK

BN = 400
GRID = CN // BN


def _sc_gather_body(x_hbm, idx_hbm, out_hbm, idx_v, rows0, rows1, gsem0, gsem1):
    sid = lax.axis_index("s")
    base = sid * (NCH * CHUNK)
    pltpu.sync_copy(idx_hbm.at[pl.ds(base, NCH * CHUNK)], idx_v)

    def gidx(g):
        return idx_v.at[pl.ds(g * CHUNK, CHUNK)]

    pltpu.async_copy(x_hbm.at[gidx(0)], rows0, gsem0)
    pltpu.async_copy(x_hbm.at[gidx(1)], rows1, gsem1)

    def iter2(i, carry):
        t = 2 * i
        for b, rows, gsem in ((0, rows0, gsem0), (1, rows1, gsem1)):
            g = t + b
            pltpu.make_async_copy(x_hbm.at[gidx(g)], rows, gsem).wait()
            pltpu.sync_copy(rows, out_hbm.at[pl.ds(base + g * CHUNK, CHUNK)])
            pltpu.async_copy(x_hbm.at[gidx(g + 2)], rows, gsem)
        return carry

    lax.fori_loop(0, (NCH - 2) // 2, iter2, 0)

    for b, rows, gsem in ((0, rows0, gsem0), (1, rows1, gsem1)):
        g = NCH - 2 + b
        pltpu.make_async_copy(x_hbm.at[gidx(g)], rows, gsem).wait()
        pltpu.sync_copy(rows, out_hbm.at[pl.ds(base + g * CHUNK, CHUNK)])


def _sc_warm_body(src_hbm, idx_hbm, out_hbm, idx_v, rows, sem):
    sid = lax.axis_index("s")
    base = sid * CHUNK
    pltpu.sync_copy(idx_hbm.at[pl.ds(base, CHUNK)], idx_v)
    pltpu.async_copy(src_hbm.at[idx_v], rows, sem).wait()
    pltpu.sync_copy(rows, out_hbm.at[pl.ds(base, CHUNK)])


@functools.cache
def _sc_warm():
    return pl.kernel(
        _sc_warm_body,
        out_type=jax.ShapeDtypeStruct((NS * CHUNK, EMB // 2), jnp.int32),
        mesh=plsc.VectorSubcoreMesh(
            core_axis_name="c", subcore_axis_name="s",
            num_cores=1, num_subcores=NS,
        ),
        scratch_types=[
            pltpu.VMEM((CHUNK,), jnp.int32),
            pltpu.VMEM((CHUNK, EMB // 2), jnp.int32),
            pltpu.SemaphoreType.DMA,
        ],
    )


@functools.cache
def _sc_gather():
    return pl.kernel(
        _sc_gather_body,
        out_type=jax.ShapeDtypeStruct((EPAD, EMB // 2), jnp.int32),
        mesh=plsc.VectorSubcoreMesh(
            core_axis_name="c", subcore_axis_name="s",
            num_cores=1, num_subcores=NS,
        ),
        scratch_types=[
            pltpu.VMEM((NCH * CHUNK,), jnp.int32),
            pltpu.VMEM((CHUNK, EMB // 2), jnp.int32),
            pltpu.VMEM((CHUNK, EMB // 2), jnp.int32),
            pltpu.SemaphoreType.DMA,
            pltpu.SemaphoreType.DMA,
        ],
    )


_DN_T = (((1,), (1,)), ((), ()))


def _pack_body(x_ref, out_ref):
    b = lax.bitcast_convert_type(x_ref[...], jnp.int32)
    bl = b[:, : EMB // 2]
    br = b[:, EMB // 2:]

    def rnd(v):
        return (v + jnp.int32(0x7FFF) + ((v >> 16) & 1)) & jnp.int32(-65536)

    out_ref[...] = lax.shift_right_logical(rnd(bl), 16) | rnd(br)


def _pack(x2):
    pbn = 1000
    return pl.pallas_call(
        _pack_body,
        grid=(N // pbn,),
        in_specs=[pl.BlockSpec((pbn, EMB), lambda i: (i, 0))],
        out_specs=pl.BlockSpec((pbn, EMB // 2), lambda i: (i, 0)),
        out_shape=jax.ShapeDtypeStruct((N, EMB // 2), jnp.int32),
    )(x2)


def _attn_body(x_ref, xg_ref, wq_ref, bq_ref, wk_ref, bkh_ref,
               wv_ref, bv_ref, out_ref, att_ref):
    f32 = jnp.float32
    x = x_ref[...]
    xgp = xg_ref[...]
    bf = jnp.bfloat16
    xg_l = lax.bitcast_convert_type(xgp << 16, f32).astype(bf)
    xg_r = lax.bitcast_convert_type(xgp & jnp.int32(-65536), f32).astype(bf)
    xg = jnp.concatenate([xg_l, xg_r], axis=1)
    q = lax.dot_general(x, wq_ref[...], _DN_T,
                        preferred_element_type=f32) + bq_ref[...]
    kg = lax.dot_general(xg, wk_ref[...], _DN_T, preferred_element_type=f32)
    vg = lax.dot_general(xg, wv_ref[...], _DN_T, preferred_element_type=f32)

    col = lax.broadcasted_iota(jnp.int32, (EMB, HEADS), 0) // DH
    head = lax.broadcasted_iota(jnp.int32, (EMB, HEADS), 1)
    hm = (col == head).astype(f32)

    qrep = jnp.broadcast_to(q[:, None, :], (BN, NNB, EMB)).reshape(BN * NNB, EMB)
    e8 = (jnp.dot(qrep * kg, hm, preferred_element_type=f32)
          + jnp.dot(qrep, bkh_ref[...], preferred_element_type=f32))
    e3 = e8.reshape(BN, NNB, HEADS)

    p = jnp.exp(e3)
    s = jnp.sum(p, axis=1, keepdims=True)
    att3 = p * (0.0625 / s)

    att_ref[...] = jnp.sum(att3, axis=(1, 2)).reshape(BN, 1)

    attexp = jnp.dot(att3.reshape(BN * NNB, HEADS), hm.T,
                     preferred_element_type=f32)
    out3 = (attexp * vg).reshape(BN, NNB, EMB)
    out_ref[...] = jnp.sum(out3, axis=1) + bv_ref[...] * 0.0625


def _attention(p, x2, xg, wq, bq2, wkb, bkh, wvb, bv2):
    off = p * GRID
    return pl.pallas_call(
        _attn_body,
        grid=(GRID,),
        in_specs=[
            pl.BlockSpec((BN, EMB), lambda i: (i + off, 0)),
            pl.BlockSpec((BN * NNB, EMB // 2), lambda i: (i, 0)),
            pl.BlockSpec((EMB, EMB), lambda i: (0, 0)),
            pl.BlockSpec((1, EMB), lambda i: (0, 0)),
            pl.BlockSpec((EMB, EMB), lambda i: (0, 0)),
            pl.BlockSpec((EMB, HEADS), lambda i: (0, 0)),
            pl.BlockSpec((EMB, EMB), lambda i: (0, 0)),
            pl.BlockSpec((1, EMB), lambda i: (0, 0)),
        ],
        out_specs=[
            pl.BlockSpec((BN, EMB), lambda i: (i, 0)),
            pl.BlockSpec((BN, 1), lambda i: (i, 0)),
        ],
        out_shape=[
            jax.ShapeDtypeStruct((CN, EMB), jnp.float32),
            jax.ShapeDtypeStruct((CN, 1), jnp.float32),
        ],
    )(x2, xg, wq, bq2, wkb, bkh, wvb, bv2)


def _attention_args(Wk, Wv, bk):
    col = lax.broadcasted_iota(jnp.int32, (EMB, HEADS), 0) // DH
    head = lax.broadcasted_iota(jnp.int32, (EMB, HEADS), 1)
    bkh = jnp.where(col == head, bk.reshape(EMB, 1), 0.0)
    return Wk.astype(jnp.bfloat16), bkh, Wv.astype(jnp.bfloat16)


def kernel(x, A, Wq, bq, Wk, bk, Wv, bv):
    b, n, e = x.shape
    x2 = x.reshape(n, e)
    idx = A.reshape(-1).astype(jnp.int32)
    bq2, bk2, bv2 = bq.reshape(1, e), bk.reshape(1, e), bv.reshape(1, e)

    xb = _pack(x2)

    idx_full = jnp.concatenate(
        [idx, jnp.zeros((EPAD - EC,), dtype=jnp.int32)])

    warm = _sc_warm()(jnp.zeros((8, EMB // 2), jnp.int32),
                      jnp.zeros((NS * CHUNK,), jnp.int32))

    xgs = []
    for p in range(P):
        idx_p = lax.slice(idx_full, (p * EC,), (p * EC + EPAD,))
        xgs.append(_sc_gather()(xb, idx_p))
    wkb, bkh, wvb = _attention_args(Wk, Wv, bk)
    outs = [
        _attention(p, x2, xgs[p], Wq, bq2, wkb, bkh, wvb, bv2)
        for p in range(P)
    ]
    l = jnp.concatenate([o[0] for o in outs])
    att = jnp.concatenate([o[1] for o in outs])
    att = att + jnp.minimum(jnp.abs(warm[0, 0]).astype(jnp.float32), 0.0)
    return l.reshape(b, n, e), att


# speedup vs baseline: 1.2743x; 1.2743x over previous
import functools

import jax
import jax.numpy as jnp
from jax import lax
from jax.experimental import pallas as pl
from jax.experimental.pallas import tpu as pltpu
from jax.experimental.pallas import tpu_sc as plsc

EMB = 256
HEADS = 8
DH = EMB // HEADS
NNB = 16
N = 10000

P = 5
CN = N // P
EC = CN * NNB

NS = 16
CHUNK = 128
NCH = 16
EPAD = NS * NCH * CHUNK

BN = 400
GRID = CN // BN


def _sc_gather_body(x_hbm, idx_hbm, out_hbm, idx_v, rows0, rows1, gsem0, gsem1):
    sid = lax.axis_index("s")
    base = sid * (NCH * CHUNK)
    pltpu.sync_copy(idx_hbm.at[pl.ds(base, NCH * CHUNK)], idx_v)

    def gidx(g):
        return idx_v.at[pl.ds(g * CHUNK, CHUNK)]

    pltpu.async_copy(x_hbm.at[gidx(0)], rows0, gsem0)
    pltpu.async_copy(x_hbm.at[gidx(1)], rows1, gsem1)

    def iter2(i, carry):
        t = 2 * i
        for b, rows, gsem in ((0, rows0, gsem0), (1, rows1, gsem1)):
            g = t + b
            pltpu.make_async_copy(x_hbm.at[gidx(g)], rows, gsem).wait()
            pltpu.sync_copy(rows, out_hbm.at[pl.ds(base + g * CHUNK, CHUNK)])
            pltpu.async_copy(x_hbm.at[gidx(g + 2)], rows, gsem)
        return carry

    lax.fori_loop(0, (NCH - 2) // 2, iter2, 0)

    for b, rows, gsem in ((0, rows0, gsem0), (1, rows1, gsem1)):
        g = NCH - 2 + b
        pltpu.make_async_copy(x_hbm.at[gidx(g)], rows, gsem).wait()
        pltpu.sync_copy(rows, out_hbm.at[pl.ds(base + g * CHUNK, CHUNK)])


@functools.cache
def _sc_gather():
    return pl.kernel(
        _sc_gather_body,
        out_type=jax.ShapeDtypeStruct((EPAD, EMB // 2), jnp.int32),
        mesh=plsc.VectorSubcoreMesh(
            core_axis_name="c", subcore_axis_name="s",
            num_cores=1, num_subcores=NS,
        ),
        scratch_types=[
            pltpu.VMEM((NCH * CHUNK,), jnp.int32),
            pltpu.VMEM((CHUNK, EMB // 2), jnp.int32),
            pltpu.VMEM((CHUNK, EMB // 2), jnp.int32),
            pltpu.SemaphoreType.DMA,
            pltpu.SemaphoreType.DMA,
        ],
    )


_DN_T = (((1,), (1,)), ((), ()))


def _pack_body(x_ref, out_ref):
    b = lax.bitcast_convert_type(x_ref[...], jnp.int32)
    bl = b[:, : EMB // 2]
    br = b[:, EMB // 2:]

    def rnd(v):
        return (v + jnp.int32(0x7FFF) + ((v >> 16) & 1)) & jnp.int32(-65536)

    out_ref[...] = lax.shift_right_logical(rnd(bl), 16) | rnd(br)


def _pack(x2):
    pbn = 1000
    return pl.pallas_call(
        _pack_body,
        grid=(N // pbn,),
        in_specs=[pl.BlockSpec((pbn, EMB), lambda i: (i, 0))],
        out_specs=pl.BlockSpec((pbn, EMB // 2), lambda i: (i, 0)),
        out_shape=jax.ShapeDtypeStruct((N, EMB // 2), jnp.int32),
    )(x2)


def _attn_body(x_ref, xg_ref, wq_ref, bq_ref, wk_ref, bkh_ref,
               wv_ref, bv_ref, out_ref, att_ref):
    f32 = jnp.float32
    x = x_ref[...]
    xgp = xg_ref[...]
    bf = jnp.bfloat16
    xg_l = lax.bitcast_convert_type(xgp << 16, f32).astype(bf)
    xg_r = lax.bitcast_convert_type(xgp & jnp.int32(-65536), f32).astype(bf)
    xg = jnp.concatenate([xg_l, xg_r], axis=1)
    q = lax.dot_general(x, wq_ref[...], _DN_T,
                        preferred_element_type=f32) + bq_ref[...]
    kg = lax.dot_general(xg, wk_ref[...], _DN_T, preferred_element_type=f32)
    vg = lax.dot_general(xg, wv_ref[...], _DN_T, preferred_element_type=f32)

    col = lax.broadcasted_iota(jnp.int32, (EMB, HEADS), 0) // DH
    head = lax.broadcasted_iota(jnp.int32, (EMB, HEADS), 1)
    hm = (col == head).astype(f32)

    qrep = jnp.broadcast_to(q[:, None, :], (BN, NNB, EMB)).reshape(BN * NNB, EMB)
    e8 = (jnp.dot(qrep * kg, hm, preferred_element_type=f32)
          + jnp.dot(qrep, bkh_ref[...], preferred_element_type=f32))
    e3 = e8.reshape(BN, NNB, HEADS)

    p = jnp.exp(e3)
    s = jnp.sum(p, axis=1, keepdims=True)
    att3 = p * (0.0625 / s)

    att_ref[...] = jnp.sum(att3, axis=(1, 2)).reshape(BN, 1)

    attexp = jnp.dot(att3.reshape(BN * NNB, HEADS), hm.T,
                     preferred_element_type=f32)
    out3 = (attexp * vg).reshape(BN, NNB, EMB)
    out_ref[...] = jnp.sum(out3, axis=1) + bv_ref[...] * 0.0625


def _attention(p, x2, xg, wq, bq2, wkb, bkh, wvb, bv2):
    off = p * GRID
    return pl.pallas_call(
        _attn_body,
        grid=(GRID,),
        in_specs=[
            pl.BlockSpec((BN, EMB), lambda i: (i + off, 0)),
            pl.BlockSpec((BN * NNB, EMB // 2), lambda i: (i, 0)),
            pl.BlockSpec((EMB, EMB), lambda i: (0, 0)),
            pl.BlockSpec((1, EMB), lambda i: (0, 0)),
            pl.BlockSpec((EMB, EMB), lambda i: (0, 0)),
            pl.BlockSpec((EMB, HEADS), lambda i: (0, 0)),
            pl.BlockSpec((EMB, EMB), lambda i: (0, 0)),
            pl.BlockSpec((1, EMB), lambda i: (0, 0)),
        ],
        out_specs=[
            pl.BlockSpec((BN, EMB), lambda i: (i, 0)),
            pl.BlockSpec((BN, 1), lambda i: (i, 0)),
        ],
        out_shape=[
            jax.ShapeDtypeStruct((CN, EMB), jnp.float32),
            jax.ShapeDtypeStruct((CN, 1), jnp.float32),
        ],
    )(x2, xg, wq, bq2, wkb, bkh, wvb, bv2)


def _attention_args(Wk, Wv, bk):
    col = lax.broadcasted_iota(jnp.int32, (EMB, HEADS), 0) // DH
    head = lax.broadcasted_iota(jnp.int32, (EMB, HEADS), 1)
    bkh = jnp.where(col == head, bk.reshape(EMB, 1), 0.0)
    return Wk.astype(jnp.bfloat16), bkh, Wv.astype(jnp.bfloat16)


def kernel(x, A, Wq, bq, Wk, bk, Wv, bv):
    b, n, e = x.shape
    x2 = x.reshape(n, e)
    idx = A.reshape(-1).astype(jnp.int32)
    bq2, bk2, bv2 = bq.reshape(1, e), bk.reshape(1, e), bv.reshape(1, e)

    xb = _pack(x2)

    idx_full = jnp.concatenate(
        [idx, jnp.zeros((EPAD - EC,), dtype=jnp.int32)])

    xgs = []
    for p in range(P):
        idx_p = lax.slice(idx_full, (p * EC,), (p * EC + EPAD,))
        xgs.append(_sc_gather()(xb, idx_p))
    wkb, bkh, wvb = _attention_args(Wk, Wv, bk)
    outs = [
        _attention(p, x2, xgs[p], Wq, bq2, wkb, bkh, wvb, bv2)
        for p in range(P)
    ]
    l = jnp.concatenate([o[0] for o in outs])
    att = jnp.concatenate([o[1] for o in outs])
    return l.reshape(b, n, e), att
